# Initial kernel scaffold; baseline (speedup 1.0000x reference)
import jax
import jax.numpy as jnp
from jax.experimental import pallas as pl


def kernel(x):
    raise NotImplementedError("write your pallas kernel here")


# speedup vs baseline: 97.7923x; 97.7923x over previous
import functools

import jax
import jax.numpy as jnp
from jax import lax
from jax.experimental import pallas as pl
from jax.experimental.pallas import tpu as pltpu
from jax.experimental.pallas import tpu_sc as plsc

_NUM_CLASSES = 4
_LANES = 16


@functools.lru_cache(maxsize=None)
def _build_sc_kernel(n_batch: int, n_pix: int, chunk: int):
    assert n_pix % chunk == 0 and chunk % _LANES == 0
    n_chunks = n_pix // chunk
    assert n_chunks >= 2
    mesh = plsc.VectorSubcoreMesh(core_axis_name="c", subcore_axis_name="s")
    n_cores = mesh.num_cores
    n_workers = n_cores * mesh.num_subcores
    assert n_batch == n_workers

    @functools.partial(
        pl.kernel,
        out_type=jax.ShapeDtypeStruct((n_batch, _NUM_CLASSES, n_pix),
                                      jnp.float32),
        mesh=mesh,
        scratch_types=[
            pltpu.VMEM((2, chunk), jnp.float32),
            pltpu.VMEM((2, _NUM_CLASSES, chunk), jnp.float32),
            pltpu.SemaphoreType.DMA,
            pltpu.SemaphoreType.DMA,
        ],
    )
    def one_hot_sc(x_hbm, out_hbm, x_v, o_v, in_sem, out_sem):
        wid = lax.axis_index("s") * n_cores + lax.axis_index("c")

        def load_desc(ci, slot):
            return pltpu.make_async_copy(
                x_hbm.at[wid, pl.ds(ci * chunk, chunk)], x_v.at[slot], in_sem)

        def store_desc(ci, slot):
            return pltpu.make_async_copy(
                o_v.at[slot], out_hbm.at[wid, :, pl.ds(ci * chunk, chunk)],
                out_sem)

        def compute(slot):
            def vec_body(i, _):
                v = x_v[slot, pl.ds(i * _LANES, _LANES)]
                f0 = jnp.where(v < 0.1, 1.0, 0.0).astype(jnp.float32)
                f1 = jnp.where(v < 1.0, 1.0, 0.0).astype(jnp.float32)
                f2 = jnp.where(v < 2.5, 1.0, 0.0).astype(jnp.float32)
                o_v[slot, 0, pl.ds(i * _LANES, _LANES)] = f0
                o_v[slot, 1, pl.ds(i * _LANES, _LANES)] = f1 - f0
                o_v[slot, 2, pl.ds(i * _LANES, _LANES)] = f2 - f1
                o_v[slot, 3, pl.ds(i * _LANES, _LANES)] = 1.0 - f2
                return 0

            lax.fori_loop(0, chunk // _LANES, vec_body, 0, unroll=4)

        load_desc(0, 0).start()
        load_desc(0, 0).wait()

        def chunk_body(ci, _):
            slot = lax.rem(ci, 2)

            @pl.when(ci + 1 < n_chunks)
            def _():
                load_desc(ci + 1, 1 - slot).start()

            @pl.when(ci >= 2)
            def _():
                store_desc(ci - 2, slot).wait()

            compute(slot)
            store_desc(ci, slot).start()

            @pl.when(ci + 1 < n_chunks)
            def _():
                load_desc(ci + 1, 1 - slot).wait()

            return 0

        lax.fori_loop(0, n_chunks, chunk_body, 0)
        store_desc(n_chunks - 2, lax.rem(n_chunks - 2, 2)).wait()
        store_desc(n_chunks - 1, lax.rem(n_chunks - 1, 2)).wait()

    return one_hot_sc


def kernel(x):
    n_batch, seq, height, width = x.shape
    n_pix = height * width
    x2 = x.reshape(n_batch, n_pix)
    one_hot_sc = _build_sc_kernel(n_batch, n_pix, chunk=8192)
    out = one_hot_sc(x2)
    return out.reshape(n_batch, seq, _NUM_CLASSES, height, width)


# speedup vs baseline: 285.4225x; 2.9187x over previous
import functools

import jax
import jax.numpy as jnp
from jax import lax
from jax.experimental import pallas as pl
from jax.experimental.pallas import tpu as pltpu
from jax.experimental.pallas import tpu_sc as plsc

_NUM_CLASSES = 4
_LANES = 16


@functools.lru_cache(maxsize=None)
def _build_sc_kernel(n_batch: int, height: int, width: int, rows: int):
    assert height % rows == 0 and width % _LANES == 0
    n_chunks = height // rows
    assert n_chunks >= 2
    vecs_per_row = width // _LANES
    mesh = plsc.VectorSubcoreMesh(core_axis_name="c", subcore_axis_name="s")
    n_cores = mesh.num_cores
    n_workers = n_cores * mesh.num_subcores
    assert n_batch == n_workers
    out_shape = (n_batch, 1, _NUM_CLASSES, height, width)

    @functools.partial(
        pl.kernel,
        out_type=jax.ShapeDtypeStruct(out_shape, jnp.float32),
        mesh=mesh,
        scratch_types=[
            pltpu.VMEM((2, rows, width), jnp.float32),
            pltpu.VMEM((2, _NUM_CLASSES, rows, width), jnp.float32),
            pltpu.SemaphoreType.DMA,
            pltpu.SemaphoreType.DMA,
        ],
    )
    def one_hot_sc(x_hbm, out_hbm, x_v, o_v, in_sem, out_sem):
        wid = lax.axis_index("s") * n_cores + lax.axis_index("c")

        def load_desc(ci, slot):
            return pltpu.make_async_copy(
                x_hbm.at[wid, 0, pl.ds(ci * rows, rows), :], x_v.at[slot],
                in_sem)

        def store_desc(ci, slot):
            return pltpu.make_async_copy(
                o_v.at[slot],
                out_hbm.at[wid, 0, :, pl.ds(ci * rows, rows), :], out_sem)

        def compute(slot):
            def row_body(r, _):
                def vec_body(i, _):
                    v = x_v[slot, r, pl.ds(i * _LANES, _LANES)]
                    f0 = jnp.where(v < 0.1, 1.0, 0.0).astype(jnp.float32)
                    f1 = jnp.where(v < 1.0, 1.0, 0.0).astype(jnp.float32)
                    f2 = jnp.where(v < 2.5, 1.0, 0.0).astype(jnp.float32)
                    o_v[slot, 0, r, pl.ds(i * _LANES, _LANES)] = f0
                    o_v[slot, 1, r, pl.ds(i * _LANES, _LANES)] = f1 - f0
                    o_v[slot, 2, r, pl.ds(i * _LANES, _LANES)] = f2 - f1
                    o_v[slot, 3, r, pl.ds(i * _LANES, _LANES)] = 1.0 - f2
                    return 0

                lax.fori_loop(0, vecs_per_row, vec_body, 0, unroll=4)
                return 0

            lax.fori_loop(0, rows, row_body, 0)

        load_desc(0, 0).start()
        load_desc(0, 0).wait()

        def chunk_body(ci, _):
            slot = lax.rem(ci, 2)

            @pl.when(ci + 1 < n_chunks)
            def _():
                load_desc(ci + 1, 1 - slot).start()

            @pl.when(ci >= 2)
            def _():
                store_desc(ci - 2, slot).wait()

            compute(slot)
            store_desc(ci, slot).start()

            @pl.when(ci + 1 < n_chunks)
            def _():
                load_desc(ci + 1, 1 - slot).wait()

            return 0

        lax.fori_loop(0, n_chunks, chunk_body, 0)
        store_desc(n_chunks - 2, lax.rem(n_chunks - 2, 2)).wait()
        store_desc(n_chunks - 1, lax.rem(n_chunks - 1, 2)).wait()

    return one_hot_sc


def kernel(x):
    n_batch, seq, height, width = x.shape
    one_hot_sc = _build_sc_kernel(n_batch, height, width, rows=16)
    return one_hot_sc(x)
